# baseline (speedup 1.0000x reference)
import functools

import jax
import jax.numpy as jnp
from jax.experimental import pallas as pl
from jax.experimental.pallas import tpu as pltpu

D_MODEL = 4096
EXPERTS = 64
TOPK = 8
BLK = 1024


def _router_tc_kernel(x_ref, w_ref, b_ref, probs_ref, idx_ref):
    x = x_ref[...]
    w = w_ref[...]
    b = b_ref[...]
    logits = jnp.dot(x, w, preferred_element_type=jnp.float32) + b

    r = logits.shape[0]

    bits = jax.lax.bitcast_convert_type(logits, jnp.int32)
    key = jnp.where(bits < 0, bits ^ jnp.int32(0x7FFFFFFF), bits)

    cur = key
    sel = jnp.zeros((r, EXPERTS), jnp.bool_)
    chosen_list = []
    m_first = None
    for k in range(TOPK):
        m = jnp.max(cur, axis=-1, keepdims=True)
        if k == 0:
            m_first = m
        chosen = cur == m
        sel = sel | chosen
        cur = jnp.where(chosen, jnp.int32(-(2**31)), cur)
        chosen_list.append(chosen.astype(jnp.float32))

    ch = jnp.concatenate(chosen_list, axis=1)
    rowi = jax.lax.broadcasted_iota(jnp.int32, (TOPK * EXPERTS, TOPK), 0)
    colk = jax.lax.broadcasted_iota(jnp.int32, (TOPK * EXPERTS, TOPK), 1)
    emat = jnp.where(rowi // EXPERTS == colk, rowi % EXPERTS, 0).astype(jnp.float32)
    idxf = jnp.dot(ch, emat, preferred_element_type=jnp.float32)
    idx_ref[...] = idxf.astype(jnp.int32)

    m0 = jax.lax.bitcast_convert_type(
        jnp.where(m_first < 0, m_first ^ jnp.int32(0x7FFFFFFF), m_first),
        jnp.float32,
    )
    e = jnp.exp(logits - m0)
    z = jnp.sum(jnp.where(sel, e, 0.0), axis=-1, keepdims=True)
    probs_ref[...] = jnp.where(sel, e / z, 0.0)


@jax.jit
def kernel(x, W_topk, b_topk, W_noisy, b_noisy):
    del W_noisy, b_noisy
    B, S, D = x.shape
    rows = B * S
    x2 = x.reshape(rows, D)
    b2 = b_topk.reshape(1, EXPERTS)

    grid = (rows // BLK,)
    probs, idx = pl.pallas_call(
        _router_tc_kernel,
        grid=grid,
        in_specs=[
            pl.BlockSpec((BLK, D), lambda i: (i, 0)),
            pl.BlockSpec((D, EXPERTS), lambda i: (0, 0)),
            pl.BlockSpec((1, EXPERTS), lambda i: (0, 0)),
        ],
        out_specs=[
            pl.BlockSpec((BLK, EXPERTS), lambda i: (i, 0)),
            pl.BlockSpec((BLK, TOPK), lambda i: (i, 0)),
        ],
        out_shape=[
            jax.ShapeDtypeStruct((rows, EXPERTS), jnp.float32),
            jax.ShapeDtypeStruct((rows, TOPK), jnp.int32),
        ],
        compiler_params=pltpu.CompilerParams(
            dimension_semantics=("arbitrary",),
        ),
    )(x2, W_topk, b2)

    return probs.reshape(B, S, EXPERTS), idx.reshape(B, S, TOPK)


# speedup vs baseline: 1.1711x; 1.1711x over previous
import functools

import jax
import jax.numpy as jnp
from jax.experimental import pallas as pl
from jax.experimental.pallas import tpu as pltpu

D_MODEL = 4096
EXPERTS = 64
TOPK = 8
BLK = 1024


def _router_tc_kernel(x_ref, w_ref, b_ref, probs_ref, idx_ref):
    x = x_ref[...]
    w = w_ref[...]
    b = b_ref[...]
    logits = jnp.dot(x, w, preferred_element_type=jnp.float32) + b

    r = logits.shape[0]

    cur = logits
    sel = jnp.zeros((r, EXPERTS), jnp.bool_)
    chosen_list = []
    m_first = None
    for k in range(TOPK):
        m = jnp.max(cur, axis=-1, keepdims=True)
        if k == 0:
            m_first = m
        chosen = cur == m
        sel = sel | chosen
        cur = jnp.where(chosen, -jnp.inf, cur)
        chosen_list.append(chosen.astype(jnp.float32))

    ch = jnp.concatenate(chosen_list, axis=1)
    rowi = jax.lax.broadcasted_iota(jnp.int32, (TOPK * EXPERTS, TOPK), 0)
    colk = jax.lax.broadcasted_iota(jnp.int32, (TOPK * EXPERTS, TOPK), 1)
    emat = jnp.where(rowi // EXPERTS == colk, rowi % EXPERTS, 0).astype(jnp.float32)
    idxf = jnp.dot(ch, emat, preferred_element_type=jnp.float32)
    idx_ref[...] = idxf.astype(jnp.int32)

    e = jnp.exp(logits - m_first)
    z = jnp.sum(jnp.where(sel, e, 0.0), axis=-1, keepdims=True)
    probs_ref[...] = jnp.where(sel, e / z, 0.0)


@jax.jit
def kernel(x, W_topk, b_topk, W_noisy, b_noisy):
    del W_noisy, b_noisy
    B, S, D = x.shape
    rows = B * S
    x2 = x.reshape(rows, D)
    b2 = b_topk.reshape(1, EXPERTS)

    grid = (rows // BLK,)
    probs, idx = pl.pallas_call(
        _router_tc_kernel,
        grid=grid,
        in_specs=[
            pl.BlockSpec((BLK, D), lambda i: (i, 0)),
            pl.BlockSpec((D, EXPERTS), lambda i: (0, 0)),
            pl.BlockSpec((1, EXPERTS), lambda i: (0, 0)),
        ],
        out_specs=[
            pl.BlockSpec((BLK, EXPERTS), lambda i: (i, 0)),
            pl.BlockSpec((BLK, TOPK), lambda i: (i, 0)),
        ],
        out_shape=[
            jax.ShapeDtypeStruct((rows, EXPERTS), jnp.float32),
            jax.ShapeDtypeStruct((rows, TOPK), jnp.int32),
        ],
        compiler_params=pltpu.CompilerParams(
            dimension_semantics=("arbitrary",),
        ),
    )(x2, W_topk, b2)

    return probs.reshape(B, S, EXPERTS), idx.reshape(B, S, TOPK)


# speedup vs baseline: 1.1889x; 1.0152x over previous
import functools

import jax
import jax.numpy as jnp
from jax.experimental import pallas as pl
from jax.experimental.pallas import tpu as pltpu

D_MODEL = 4096
EXPERTS = 64
TOPK = 8
BLK = 512


def _router_tc_kernel(x_ref, w_ref, b_ref, probs_ref, idx_ref):
    x = x_ref[...]
    w = w_ref[...]
    b = b_ref[...]
    logits = jnp.dot(x, w, preferred_element_type=jnp.float32) + b

    r = logits.shape[0]

    cur = logits
    sel = jnp.zeros((r, EXPERTS), jnp.bool_)
    chosen_list = []
    m_first = None
    for k in range(TOPK):
        m = jnp.max(cur, axis=-1, keepdims=True)
        if k == 0:
            m_first = m
        chosen = cur == m
        sel = sel | chosen
        cur = jnp.where(chosen, -jnp.inf, cur)
        chosen_list.append(chosen.astype(jnp.float32))

    ch = jnp.concatenate(chosen_list, axis=1)
    rowi = jax.lax.broadcasted_iota(jnp.int32, (TOPK * EXPERTS, TOPK), 0)
    colk = jax.lax.broadcasted_iota(jnp.int32, (TOPK * EXPERTS, TOPK), 1)
    emat = jnp.where(rowi // EXPERTS == colk, rowi % EXPERTS, 0).astype(jnp.float32)
    idxf = jnp.dot(ch, emat, preferred_element_type=jnp.float32)
    idx_ref[...] = idxf.astype(jnp.int32)

    e = jnp.exp(logits - m_first)
    z = jnp.sum(jnp.where(sel, e, 0.0), axis=-1, keepdims=True)
    probs_ref[...] = jnp.where(sel, e / z, 0.0)


@jax.jit
def kernel(x, W_topk, b_topk, W_noisy, b_noisy):
    del W_noisy, b_noisy
    B, S, D = x.shape
    rows = B * S
    x2 = x.reshape(rows, D)
    b2 = b_topk.reshape(1, EXPERTS)

    grid = (rows // BLK,)
    probs, idx = pl.pallas_call(
        _router_tc_kernel,
        grid=grid,
        in_specs=[
            pl.BlockSpec((BLK, D), lambda i: (i, 0)),
            pl.BlockSpec((D, EXPERTS), lambda i: (0, 0)),
            pl.BlockSpec((1, EXPERTS), lambda i: (0, 0)),
        ],
        out_specs=[
            pl.BlockSpec((BLK, EXPERTS), lambda i: (i, 0)),
            pl.BlockSpec((BLK, TOPK), lambda i: (i, 0)),
        ],
        out_shape=[
            jax.ShapeDtypeStruct((rows, EXPERTS), jnp.float32),
            jax.ShapeDtypeStruct((rows, TOPK), jnp.int32),
        ],
        compiler_params=pltpu.CompilerParams(
            dimension_semantics=("arbitrary",),
        ),
    )(x2, W_topk, b2)

    return probs.reshape(B, S, EXPERTS), idx.reshape(B, S, TOPK)
